# baseline (speedup 1.0000x reference)
import functools

import jax
import jax.numpy as jnp
from jax import lax
from jax.experimental import pallas as pl
from jax.experimental.pallas import tpu as pltpu
from jax.experimental.pallas import tpu_sc as plsc

N = 10000
E = 320000
D = 128

NC = 2
NS = 16
NW = NC * NS

NP = 10240
RPT = NP // NS
K = 112
NBUF = 3
NB = NBUF * (-(-E // (NW * K * NBUF)))
NB0 = 72
NB1 = 2 * NB - NB0
EPT0 = NB0 * K
EPT1 = NB1 * K
EP = NS * (EPT0 + EPT1)
WCH = 80

_mesh = plsc.VectorSubcoreMesh(core_axis_name="c", subcore_axis_name="s",
                               num_cores=NC, num_subcores=NS)


@functools.partial(
    pl.kernel,
    out_type=jax.ShapeDtypeStruct((NC, NP), jnp.float32),
    mesh=_mesh,
    scratch_types=[
        pltpu.VMEM_SHARED((NP,), jnp.float32),
        pltpu.VMEM((NB, K), jnp.int32),
        pltpu.VMEM((NB, K), jnp.float32),
        pltpu.SemaphoreType.DMA,
    ],
)
def _deg(col_hbm, ew_hbm, z1_hbm, out_hbm, dacc, coli, ewb, sem):
    c = lax.axis_index("c")
    s = lax.axis_index("s")
    wid = s * NC + c
    pltpu.sync_copy(col_hbm.at[wid], coli)
    pltpu.sync_copy(ew_hbm.at[wid], ewb)
    pltpu.sync_copy(z1_hbm, dacc.at[pl.ds(s * RPT, RPT)])
    plsc.subcore_barrier()

    def fire(b, carry):
        pltpu.async_copy(ewb.at[b], dacc.at[coli.at[b]], sem, add=True)
        return carry

    lax.fori_loop(0, NB, fire, 0)

    def drain(b, carry):
        pltpu.make_async_copy(ewb.at[0], dacc.at[coli.at[0]], sem).wait()
        return carry

    lax.fori_loop(0, NB, drain, 0)
    plsc.subcore_barrier()
    pltpu.sync_copy(dacc.at[pl.ds(s * RPT, RPT)],
                    out_hbm.at[c, pl.ds(s * RPT, RPT)])


@functools.partial(
    pl.kernel,
    out_type=jax.ShapeDtypeStruct((NC, NP, D), jnp.float32),
    mesh=_mesh,
    scratch_types=[
        pltpu.VMEM_SHARED((NP, D), jnp.float32),
        pltpu.VMEM((NBUF, K), jnp.int32),
        pltpu.VMEM((NBUF, K), jnp.int32),
        pltpu.VMEM((NBUF, K), jnp.float32),
        pltpu.VMEM((NBUF, K, D), jnp.float32),
        pltpu.VMEM((D,), jnp.float32),
        pltpu.VMEM((WCH,), jnp.float32),
        pltpu.SemaphoreType.DMA((NBUF,)),
        pltpu.SemaphoreType.DMA((NBUF,)),
        pltpu.SemaphoreType.DMA((NBUF,)),
    ],
)
def _conv(g_hbm, dinv_hbm, row_hbm, col_hbm, ew_hbm, bh_hbm, z2_hbm, out_hbm,
          acc, rowi, coli, ewb, rowsb, b_v, dc_v, isem, gsem, ssem):
    c = lax.axis_index("c")
    s = lax.axis_index("s")
    tbase = jnp.where(c == 0, s * EPT0, NS * EPT0 + s * EPT1)
    nb = jnp.where(c == 0, NB0, NB1)
    pltpu.sync_copy(bh_hbm, b_v)
    pltpu.sync_copy(z2_hbm, acc.at[pl.ds(s * RPT, RPT)])

    def issue_idx(b, i):
        base = tbase + b * K
        pltpu.async_copy(row_hbm.at[pl.ds(base, K)], rowi.at[i], isem.at[i])
        pltpu.async_copy(col_hbm.at[pl.ds(base, K)], coli.at[i], isem.at[i])
        pltpu.async_copy(ew_hbm.at[pl.ds(base, K)], ewb.at[i], isem.at[i])

    def wait_idx(i):
        pltpu.make_async_copy(row_hbm.at[pl.ds(0, K)], rowi.at[i],
                              isem.at[i]).wait()
        pltpu.make_async_copy(col_hbm.at[pl.ds(0, K)], coli.at[i],
                              isem.at[i]).wait()
        pltpu.make_async_copy(ew_hbm.at[pl.ds(0, K)], ewb.at[i],
                              isem.at[i]).wait()

    def issue_gather(i):
        pltpu.async_copy(g_hbm.at[rowi.at[i]], rowsb.at[i], gsem.at[i])

    def wait_gather(i):
        pltpu.make_async_copy(g_hbm.at[rowi.at[0]], rowsb.at[i],
                              gsem.at[i]).wait()

    def issue_scatter(i):
        pltpu.async_copy(rowsb.at[i], acc.at[coli.at[i]], ssem.at[i],
                         add=True)

    def wait_scatter(i):
        pltpu.make_async_copy(rowsb.at[i], acc.at[coli.at[0]],
                              ssem.at[i]).wait()

    def scale(i):
        def group(t, carry2):
            w = ewb[i, pl.ds(t * 16, 16)]
            for u in range(16):
                fv = jnp.full((16,), w[u], dtype=jnp.float32)
                k = t * 16 + u
                for j in range(D // 16):
                    sl = pl.ds(j * 16, 16)
                    rowsb[i, k, sl] = rowsb[i, k, sl] * fv
            return carry2

        lax.fori_loop(0, K // 16, group, 0)

    if NBUF == 4:
        issue_idx(0, 0)
        issue_idx(1, 1)
        issue_idx(2, 2)
        plsc.subcore_barrier()
        wait_idx(0)
        issue_gather(0)
        wait_idx(1)
        issue_gather(1)

        def outer(go, carry):
            for i in range(NBUF):
                b = go * NBUF + i
                j2 = (i + 2) % NBUF
                j3 = (i + 3) % NBUF

                @pl.when(b + 2 < nb)
                def _():
                    wait_idx(j2)
                    issue_gather(j2)

                wait_gather(i)
                scale(i)
                issue_scatter(i)

                @pl.when(b + 3 < nb)
                def _():
                    @pl.when(b >= 1)
                    def _():
                        wait_scatter(j3)

                    issue_idx(b + 3, j3)

            return carry
    else:
        issue_idx(0, 0)
        issue_idx(1, 1)
        plsc.subcore_barrier()
        wait_idx(0)
        issue_gather(0)

        def outer(go, carry):
            for i in range(NBUF):
                b = go * NBUF + i
                j1 = (i + 1) % NBUF
                j2 = (i + 2) % NBUF

                @pl.when(b + 1 < nb)
                def _():
                    wait_idx(j1)
                    issue_gather(j1)

                wait_gather(i)
                scale(i)
                issue_scatter(i)

                @pl.when(b + 2 < nb)
                def _():
                    @pl.when(b >= 1)
                    def _():
                        wait_scatter(j2)

                    issue_idx(b + 2, j2)

            return carry

    lax.fori_loop(0, nb // NBUF, outer, 0)
    for i in range(NBUF):
        wait_scatter(i)
    plsc.subcore_barrier()

    for ch in range(RPT // WCH):
        r0 = s * RPT + ch * WCH
        pltpu.sync_copy(acc.at[pl.ds(r0, WCH)], rowsb.at[0, pl.ds(0, WCH)])
        pltpu.sync_copy(dinv_hbm.at[pl.ds(r0, WCH)], dc_v)

        def wgroup(t, carry):
            dvec = dc_v[pl.ds(t * 16, 16)]
            for u in range(16):
                dv = jnp.full((16,), dvec[u], dtype=jnp.float32)
                k = t * 16 + u
                for j in range(D // 16):
                    sl = pl.ds(j * 16, 16)
                    rowsb[0, k, sl] = rowsb[0, k, sl] * dv + b_v[sl]
            return carry

        lax.fori_loop(0, WCH // 16, wgroup, 0)
        pltpu.sync_copy(rowsb.at[0, pl.ds(0, WCH)],
                        out_hbm.at[c, pl.ds(r0, WCH)])


def _k1_body(deg_ref, x_ref, we_ref, w1_ref, g1_ref, dinv_ref):
    h = jnp.maximum(jnp.dot(x_ref[...], we_ref[...],
                            preferred_element_type=jnp.float32), 0.0)
    d = deg_ref[0] + deg_ref[1]
    dinv = jnp.where(d > 0, lax.rsqrt(jnp.maximum(d, 1e-12)), 0.0)
    dinv_ref[...] = dinv
    g1_ref[...] = dinv * jnp.dot(h, w1_ref[...],
                                 preferred_element_type=jnp.float32)


def _k2_body(p_ref, w3_ref, dinv_ref, g3_ref):
    h = jnp.maximum(p_ref[0] + p_ref[1], 0.0)
    g3_ref[...] = dinv_ref[...] * jnp.dot(
        h, w3_ref[...], preferred_element_type=jnp.float32)


def _k3_body(q_ref, o_ref):
    o_ref[...] = q_ref[0] + q_ref[1]


_RB = 1024
_GRID = NP // _RB


def _tc_k1(degp, x_pad, We, W1):
    return pl.pallas_call(
        _k1_body,
        grid=(_GRID,),
        in_specs=[
            pl.BlockSpec((2, _RB, 1), lambda i: (0, i, 0)),
            pl.BlockSpec((_RB, D), lambda i: (i, 0)),
            pl.BlockSpec((D, D), lambda i: (0, 0)),
            pl.BlockSpec((D, D), lambda i: (0, 0)),
        ],
        out_specs=[
            pl.BlockSpec((_RB, D), lambda i: (i, 0)),
            pl.BlockSpec((_RB, 1), lambda i: (i, 0)),
        ],
        out_shape=[
            jax.ShapeDtypeStruct((NP, D), jnp.float32),
            jax.ShapeDtypeStruct((NP, 1), jnp.float32),
        ],
    )(degp, x_pad, We, W1)


def _tc_k2(p, W3, dinv_p):
    return pl.pallas_call(
        _k2_body,
        grid=(_GRID,),
        in_specs=[
            pl.BlockSpec((2, _RB, D), lambda i: (0, i, 0)),
            pl.BlockSpec((D, D), lambda i: (0, 0)),
            pl.BlockSpec((_RB, 1), lambda i: (i, 0)),
        ],
        out_specs=pl.BlockSpec((_RB, D), lambda i: (i, 0)),
        out_shape=jax.ShapeDtypeStruct((NP, D), jnp.float32),
    )(p, W3, dinv_p)


def _tc_k3(q):
    return pl.pallas_call(
        _k3_body,
        grid=(_GRID,),
        in_specs=[pl.BlockSpec((2, _RB, D), lambda i: (0, i, 0))],
        out_specs=pl.BlockSpec((_RB, D), lambda i: (i, 0)),
        out_shape=jax.ShapeDtypeStruct((NP, D), jnp.float32),
    )(q)


def kernel(x, edge_index, edge_weight, W_embed, W1, b1, W3, b3):
    row = edge_index[0]
    col = edge_index[1]
    padE = EP - E
    pad_col = (jnp.arange(padE, dtype=jnp.int32) * 8) % N
    row_p = jnp.concatenate([row, jnp.zeros((padE,), jnp.int32)])
    col_p = jnp.concatenate([col, pad_col])
    ew_p = jnp.concatenate([edge_weight, jnp.zeros((padE,), jnp.float32)])
    col_r = col_p.reshape(NW, NB, K)
    ew_r = ew_p.reshape(NW, NB, K)
    x_pad = jnp.pad(x, ((0, NP - N), (0, 0)))
    z1 = jnp.zeros((RPT,), jnp.float32)
    z2 = jnp.zeros((RPT, D), jnp.float32)

    degp = _deg(col_r, ew_r, z1)
    g1, dinv_p = _tc_k1(degp.reshape(2, NP, 1), x_pad, W_embed, W1)
    dinv = dinv_p.reshape(NP)
    p = _conv(g1, dinv, row_p, col_p, ew_p, 0.5 * b1, z2)
    g3 = _tc_k2(p, W3, dinv_p)
    q = _conv(g3, dinv, row_p, col_p, ew_p, 0.5 * b3, z2)
    out = _tc_k3(q)
    return out[:N]


# speedup vs baseline: 1.1307x; 1.1307x over previous
import functools

import jax
import jax.numpy as jnp
from jax import lax
from jax.experimental import pallas as pl
from jax.experimental.pallas import tpu as pltpu
from jax.experimental.pallas import tpu_sc as plsc

N = 10000
E = 320000
D = 128

NC = 2
NS = 16
NW = NC * NS

NP = 10240
RPT = NP // NS
K = 112
NBUF = 3
NB = NBUF * (-(-E // (NW * K * NBUF)))
NB0 = 108
NB1 = 2 * NB - NB0
EPT0 = NB0 * K
EPT1 = NB1 * K
EP = NS * (EPT0 + EPT1)
WCH = 80

_mesh = plsc.VectorSubcoreMesh(core_axis_name="c", subcore_axis_name="s",
                               num_cores=NC, num_subcores=NS)


@functools.partial(
    pl.kernel,
    out_type=jax.ShapeDtypeStruct((NC, NP), jnp.float32),
    mesh=_mesh,
    scratch_types=[
        pltpu.VMEM_SHARED((NP,), jnp.float32),
        pltpu.VMEM((NB, K), jnp.int32),
        pltpu.VMEM((NB, K), jnp.float32),
        pltpu.SemaphoreType.DMA,
    ],
)
def _deg(col_hbm, ew_hbm, z1_hbm, out_hbm, dacc, coli, ewb, sem):
    c = lax.axis_index("c")
    s = lax.axis_index("s")
    wid = s * NC + c
    pltpu.sync_copy(col_hbm.at[wid], coli)
    pltpu.sync_copy(ew_hbm.at[wid], ewb)
    pltpu.sync_copy(z1_hbm, dacc.at[pl.ds(s * RPT, RPT)])
    plsc.subcore_barrier()

    def fire(b, carry):
        pltpu.async_copy(ewb.at[b], dacc.at[coli.at[b]], sem, add=True)
        return carry

    lax.fori_loop(0, NB, fire, 0)

    def drain(b, carry):
        pltpu.make_async_copy(ewb.at[0], dacc.at[coli.at[0]], sem).wait()
        return carry

    lax.fori_loop(0, NB, drain, 0)
    plsc.subcore_barrier()
    pltpu.sync_copy(dacc.at[pl.ds(s * RPT, RPT)],
                    out_hbm.at[c, pl.ds(s * RPT, RPT)])


@functools.partial(
    pl.kernel,
    out_type=jax.ShapeDtypeStruct((NC, NP, D), jnp.float32),
    mesh=_mesh,
    scratch_types=[
        pltpu.VMEM_SHARED((NP, D), jnp.float32),
        pltpu.VMEM((NBUF, K), jnp.int32),
        pltpu.VMEM((NBUF, K), jnp.int32),
        pltpu.VMEM((NBUF, K), jnp.float32),
        pltpu.VMEM((NBUF, K, D), jnp.float32),
        pltpu.VMEM((D,), jnp.float32),
        pltpu.VMEM((WCH,), jnp.float32),
        pltpu.SemaphoreType.DMA((NBUF,)),
        pltpu.SemaphoreType.DMA((NBUF,)),
        pltpu.SemaphoreType.DMA((NBUF,)),
    ],
)
def _conv(g_hbm, dinv_hbm, row_hbm, col_hbm, ew_hbm, bh_hbm, z2_hbm, out_hbm,
          acc, rowi, coli, ewb, rowsb, b_v, dc_v, isem, gsem, ssem):
    c = lax.axis_index("c")
    s = lax.axis_index("s")
    tbase = jnp.where(c == 0, s * EPT0, NS * EPT0 + s * EPT1)
    nb = jnp.where(c == 0, NB0, NB1)
    pltpu.sync_copy(bh_hbm, b_v)
    pltpu.sync_copy(z2_hbm, acc.at[pl.ds(s * RPT, RPT)])

    def issue_idx(b, i):
        base = tbase + b * K
        pltpu.async_copy(row_hbm.at[pl.ds(base, K)], rowi.at[i], isem.at[i])
        pltpu.async_copy(col_hbm.at[pl.ds(base, K)], coli.at[i], isem.at[i])
        pltpu.async_copy(ew_hbm.at[pl.ds(base, K)], ewb.at[i], isem.at[i])

    def wait_idx(i):
        pltpu.make_async_copy(row_hbm.at[pl.ds(0, K)], rowi.at[i],
                              isem.at[i]).wait()
        pltpu.make_async_copy(col_hbm.at[pl.ds(0, K)], coli.at[i],
                              isem.at[i]).wait()
        pltpu.make_async_copy(ew_hbm.at[pl.ds(0, K)], ewb.at[i],
                              isem.at[i]).wait()

    def issue_gather(i):
        pltpu.async_copy(g_hbm.at[rowi.at[i]], rowsb.at[i], gsem.at[i])

    def wait_gather(i):
        pltpu.make_async_copy(g_hbm.at[rowi.at[0]], rowsb.at[i],
                              gsem.at[i]).wait()

    def issue_scatter(i):
        pltpu.async_copy(rowsb.at[i], acc.at[coli.at[i]], ssem.at[i],
                         add=True)

    def wait_scatter(i):
        pltpu.make_async_copy(rowsb.at[i], acc.at[coli.at[0]],
                              ssem.at[i]).wait()

    def scale(i):
        def group(t, carry2):
            w = ewb[i, pl.ds(t * 16, 16)]
            for u in range(16):
                fv = jnp.full((16,), w[u], dtype=jnp.float32)
                k = t * 16 + u
                for j in range(D // 16):
                    sl = pl.ds(j * 16, 16)
                    rowsb[i, k, sl] = rowsb[i, k, sl] * fv
            return carry2

        lax.fori_loop(0, K // 16, group, 0)

    if NBUF == 4:
        issue_idx(0, 0)
        issue_idx(1, 1)
        issue_idx(2, 2)
        plsc.subcore_barrier()
        wait_idx(0)
        issue_gather(0)
        wait_idx(1)
        issue_gather(1)

        def outer(go, carry):
            for i in range(NBUF):
                b = go * NBUF + i
                j2 = (i + 2) % NBUF
                j3 = (i + 3) % NBUF

                @pl.when(b + 2 < nb)
                def _():
                    wait_idx(j2)
                    issue_gather(j2)

                wait_gather(i)
                scale(i)
                issue_scatter(i)

                @pl.when(b + 3 < nb)
                def _():
                    @pl.when(b >= 1)
                    def _():
                        wait_scatter(j3)

                    issue_idx(b + 3, j3)

            return carry
    else:
        issue_idx(0, 0)
        issue_idx(1, 1)
        plsc.subcore_barrier()
        wait_idx(0)
        issue_gather(0)

        def outer(go, carry):
            for i in range(NBUF):
                b = go * NBUF + i
                j1 = (i + 1) % NBUF
                j2 = (i + 2) % NBUF

                @pl.when(b + 1 < nb)
                def _():
                    wait_idx(j1)
                    issue_gather(j1)

                wait_gather(i)
                scale(i)
                issue_scatter(i)

                @pl.when(b + 2 < nb)
                def _():
                    @pl.when(b >= 1)
                    def _():
                        wait_scatter(j2)

                    issue_idx(b + 2, j2)

            return carry

    lax.fori_loop(0, nb // NBUF, outer, 0)
    for i in range(NBUF):
        wait_scatter(i)
    plsc.subcore_barrier()

    for ch in range(RPT // WCH):
        r0 = s * RPT + ch * WCH
        pltpu.sync_copy(acc.at[pl.ds(r0, WCH)], rowsb.at[0, pl.ds(0, WCH)])
        pltpu.sync_copy(dinv_hbm.at[pl.ds(r0, WCH)], dc_v)

        def wgroup(t, carry):
            dvec = dc_v[pl.ds(t * 16, 16)]
            for u in range(16):
                dv = jnp.full((16,), dvec[u], dtype=jnp.float32)
                k = t * 16 + u
                for j in range(D // 16):
                    sl = pl.ds(j * 16, 16)
                    rowsb[0, k, sl] = rowsb[0, k, sl] * dv + b_v[sl]
            return carry

        lax.fori_loop(0, WCH // 16, wgroup, 0)
        pltpu.sync_copy(rowsb.at[0, pl.ds(0, WCH)],
                        out_hbm.at[c, pl.ds(r0, WCH)])


def _k1_body(deg_ref, x_ref, we_ref, w1_ref, g1_ref, dinv_ref):
    h = jnp.maximum(jnp.dot(x_ref[...], we_ref[...],
                            preferred_element_type=jnp.float32), 0.0)
    d = deg_ref[0] + deg_ref[1]
    dinv = jnp.where(d > 0, lax.rsqrt(jnp.maximum(d, 1e-12)), 0.0)
    dinv_ref[...] = dinv
    g1_ref[...] = dinv * jnp.dot(h, w1_ref[...],
                                 preferred_element_type=jnp.float32)


def _k2_body(p_ref, w3_ref, dinv_ref, g3_ref):
    h = jnp.maximum(p_ref[0] + p_ref[1], 0.0)
    g3_ref[...] = dinv_ref[...] * jnp.dot(
        h, w3_ref[...], preferred_element_type=jnp.float32)


def _k3_body(q_ref, o_ref):
    o_ref[...] = q_ref[0] + q_ref[1]


_RB = 1024
_GRID = NP // _RB


def _tc_k1(degp, x_pad, We, W1):
    return pl.pallas_call(
        _k1_body,
        grid=(_GRID,),
        in_specs=[
            pl.BlockSpec((2, _RB, 1), lambda i: (0, i, 0)),
            pl.BlockSpec((_RB, D), lambda i: (i, 0)),
            pl.BlockSpec((D, D), lambda i: (0, 0)),
            pl.BlockSpec((D, D), lambda i: (0, 0)),
        ],
        out_specs=[
            pl.BlockSpec((_RB, D), lambda i: (i, 0)),
            pl.BlockSpec((_RB, 1), lambda i: (i, 0)),
        ],
        out_shape=[
            jax.ShapeDtypeStruct((NP, D), jnp.float32),
            jax.ShapeDtypeStruct((NP, 1), jnp.float32),
        ],
    )(degp, x_pad, We, W1)


def _tc_k2(p, W3, dinv_p):
    return pl.pallas_call(
        _k2_body,
        grid=(_GRID,),
        in_specs=[
            pl.BlockSpec((2, _RB, D), lambda i: (0, i, 0)),
            pl.BlockSpec((D, D), lambda i: (0, 0)),
            pl.BlockSpec((_RB, 1), lambda i: (i, 0)),
        ],
        out_specs=pl.BlockSpec((_RB, D), lambda i: (i, 0)),
        out_shape=jax.ShapeDtypeStruct((NP, D), jnp.float32),
    )(p, W3, dinv_p)


def _tc_k3(q):
    return pl.pallas_call(
        _k3_body,
        grid=(_GRID,),
        in_specs=[pl.BlockSpec((2, _RB, D), lambda i: (0, i, 0))],
        out_specs=pl.BlockSpec((_RB, D), lambda i: (i, 0)),
        out_shape=jax.ShapeDtypeStruct((NP, D), jnp.float32),
    )(q)


def kernel(x, edge_index, edge_weight, W_embed, W1, b1, W3, b3):
    row = edge_index[0]
    col = edge_index[1]
    padE = EP - E
    pad_col = (jnp.arange(padE, dtype=jnp.int32) * 8) % N
    row_p = jnp.concatenate([row, jnp.zeros((padE,), jnp.int32)])
    col_p = jnp.concatenate([col, pad_col])
    ew_p = jnp.concatenate([edge_weight, jnp.zeros((padE,), jnp.float32)])
    col_r = col_p.reshape(NW, NB, K)
    ew_r = ew_p.reshape(NW, NB, K)
    x_pad = jnp.pad(x, ((0, NP - N), (0, 0)))
    z1 = jnp.zeros((RPT,), jnp.float32)
    z2 = jnp.zeros((RPT, D), jnp.float32)

    degp = _deg(col_r, ew_r, z1)
    g1, dinv_p = _tc_k1(degp.reshape(2, NP, 1), x_pad, W_embed, W1)
    dinv = dinv_p.reshape(NP)
    p = _conv(g1, dinv, row_p, col_p, ew_p, 0.5 * b1, z2)
    g3 = _tc_k2(p, W3, dinv_p)
    q = _conv(g3, dinv, row_p, col_p, ew_p, 0.5 * b3, z2)
    out = _tc_k3(q)
    return out[:N]


# speedup vs baseline: 1.1437x; 1.0115x over previous
import functools

import jax
import jax.numpy as jnp
from jax import lax
from jax.experimental import pallas as pl
from jax.experimental.pallas import tpu as pltpu
from jax.experimental.pallas import tpu_sc as plsc

N = 10000
E = 320000
D = 128

NC = 2
NS = 16
NW = NC * NS

NP = 10240
RPT = NP // NS
K = 112
NBUF = 3
NB = NBUF * (-(-E // (NW * K * NBUF)))
NB0 = 111
NB1 = 2 * NB - NB0
EPT0 = NB0 * K
EPT1 = NB1 * K
EP = NS * (EPT0 + EPT1)
WCH = 80

_mesh = plsc.VectorSubcoreMesh(core_axis_name="c", subcore_axis_name="s",
                               num_cores=NC, num_subcores=NS)


@functools.partial(
    pl.kernel,
    out_type=jax.ShapeDtypeStruct((NC, NP), jnp.float32),
    mesh=_mesh,
    scratch_types=[
        pltpu.VMEM_SHARED((NP,), jnp.float32),
        pltpu.VMEM((NB, K), jnp.int32),
        pltpu.VMEM((NB, K), jnp.float32),
        pltpu.SemaphoreType.DMA,
    ],
)
def _deg(col_hbm, ew_hbm, z1_hbm, out_hbm, dacc, coli, ewb, sem):
    c = lax.axis_index("c")
    s = lax.axis_index("s")
    wid = s * NC + c
    pltpu.sync_copy(col_hbm.at[wid], coli)
    pltpu.sync_copy(ew_hbm.at[wid], ewb)
    pltpu.sync_copy(z1_hbm, dacc.at[pl.ds(s * RPT, RPT)])
    plsc.subcore_barrier()

    def fire(b, carry):
        pltpu.async_copy(ewb.at[b], dacc.at[coli.at[b]], sem, add=True)
        return carry

    lax.fori_loop(0, NB, fire, 0)

    def drain(b, carry):
        pltpu.make_async_copy(ewb.at[0], dacc.at[coli.at[0]], sem).wait()
        return carry

    lax.fori_loop(0, NB, drain, 0)
    plsc.subcore_barrier()
    pltpu.sync_copy(dacc.at[pl.ds(s * RPT, RPT)],
                    out_hbm.at[c, pl.ds(s * RPT, RPT)])


@functools.partial(
    pl.kernel,
    out_type=jax.ShapeDtypeStruct((NC, NP, D), jnp.float32),
    mesh=_mesh,
    scratch_types=[
        pltpu.VMEM_SHARED((NP, D), jnp.float32),
        pltpu.VMEM((NBUF, K), jnp.int32),
        pltpu.VMEM((NBUF, K), jnp.int32),
        pltpu.VMEM((NBUF, K), jnp.float32),
        pltpu.VMEM((NBUF, K, D), jnp.float32),
        pltpu.VMEM((D,), jnp.float32),
        pltpu.VMEM((WCH,), jnp.float32),
        pltpu.SemaphoreType.DMA((NBUF,)),
        pltpu.SemaphoreType.DMA((NBUF,)),
        pltpu.SemaphoreType.DMA((NBUF,)),
    ],
)
def _conv(g_hbm, dinv_hbm, row_hbm, col_hbm, ew_hbm, bh_hbm, z2_hbm, out_hbm,
          acc, rowi, coli, ewb, rowsb, b_v, dc_v, isem, gsem, ssem):
    c = lax.axis_index("c")
    s = lax.axis_index("s")
    tbase = jnp.where(c == 0, s * EPT0, NS * EPT0 + s * EPT1)
    nb = jnp.where(c == 0, NB0, NB1)
    pltpu.sync_copy(bh_hbm, b_v)
    pltpu.sync_copy(z2_hbm, acc.at[pl.ds(s * RPT, RPT)])

    def issue_idx(b, i):
        base = tbase + b * K
        pltpu.async_copy(row_hbm.at[pl.ds(base, K)], rowi.at[i], isem.at[i])
        pltpu.async_copy(col_hbm.at[pl.ds(base, K)], coli.at[i], isem.at[i])
        pltpu.async_copy(ew_hbm.at[pl.ds(base, K)], ewb.at[i], isem.at[i])

    def wait_idx(i):
        pltpu.make_async_copy(row_hbm.at[pl.ds(0, K)], rowi.at[i],
                              isem.at[i]).wait()
        pltpu.make_async_copy(col_hbm.at[pl.ds(0, K)], coli.at[i],
                              isem.at[i]).wait()
        pltpu.make_async_copy(ew_hbm.at[pl.ds(0, K)], ewb.at[i],
                              isem.at[i]).wait()

    def issue_gather(i):
        pltpu.async_copy(g_hbm.at[rowi.at[i]], rowsb.at[i], gsem.at[i])

    def wait_gather(i):
        pltpu.make_async_copy(g_hbm.at[rowi.at[0]], rowsb.at[i],
                              gsem.at[i]).wait()

    def issue_scatter(i):
        pltpu.async_copy(rowsb.at[i], acc.at[coli.at[i]], ssem.at[i],
                         add=True)

    def wait_scatter(i):
        pltpu.make_async_copy(rowsb.at[i], acc.at[coli.at[0]],
                              ssem.at[i]).wait()

    def scale(i):
        def group(t, carry2):
            w = ewb[i, pl.ds(t * 16, 16)]
            for u in range(16):
                fv = jnp.full((16,), w[u], dtype=jnp.float32)
                k = t * 16 + u
                for j in range(D // 16):
                    sl = pl.ds(j * 16, 16)
                    rowsb[i, k, sl] = rowsb[i, k, sl] * fv
            return carry2

        lax.fori_loop(0, K // 16, group, 0)

    if NBUF == 4:
        issue_idx(0, 0)
        issue_idx(1, 1)
        issue_idx(2, 2)
        plsc.subcore_barrier()
        wait_idx(0)
        issue_gather(0)
        wait_idx(1)
        issue_gather(1)

        def outer(go, carry):
            for i in range(NBUF):
                b = go * NBUF + i
                j2 = (i + 2) % NBUF
                j3 = (i + 3) % NBUF

                @pl.when(b + 2 < nb)
                def _():
                    wait_idx(j2)
                    issue_gather(j2)

                wait_gather(i)
                scale(i)
                issue_scatter(i)

                @pl.when(b + 3 < nb)
                def _():
                    @pl.when(b >= 1)
                    def _():
                        wait_scatter(j3)

                    issue_idx(b + 3, j3)

            return carry
    else:
        issue_idx(0, 0)
        issue_idx(1, 1)
        plsc.subcore_barrier()
        wait_idx(0)
        issue_gather(0)

        def outer(go, carry):
            for i in range(NBUF):
                b = go * NBUF + i
                j1 = (i + 1) % NBUF
                j2 = (i + 2) % NBUF

                @pl.when(b + 1 < nb)
                def _():
                    wait_idx(j1)
                    issue_gather(j1)

                wait_gather(i)
                scale(i)
                issue_scatter(i)

                @pl.when(b + 2 < nb)
                def _():
                    @pl.when(b >= 1)
                    def _():
                        wait_scatter(j2)

                    issue_idx(b + 2, j2)

            return carry

    lax.fori_loop(0, nb // NBUF, outer, 0)
    for i in range(NBUF):
        wait_scatter(i)
    plsc.subcore_barrier()

    for ch in range(RPT // WCH):
        r0 = s * RPT + ch * WCH
        pltpu.sync_copy(acc.at[pl.ds(r0, WCH)], rowsb.at[0, pl.ds(0, WCH)])
        pltpu.sync_copy(dinv_hbm.at[pl.ds(r0, WCH)], dc_v)

        def wgroup(t, carry):
            dvec = dc_v[pl.ds(t * 16, 16)]
            for u in range(16):
                dv = jnp.full((16,), dvec[u], dtype=jnp.float32)
                k = t * 16 + u
                for j in range(D // 16):
                    sl = pl.ds(j * 16, 16)
                    rowsb[0, k, sl] = rowsb[0, k, sl] * dv + b_v[sl]
            return carry

        lax.fori_loop(0, WCH // 16, wgroup, 0)
        pltpu.sync_copy(rowsb.at[0, pl.ds(0, WCH)],
                        out_hbm.at[c, pl.ds(r0, WCH)])


def _k1_body(deg_ref, x_ref, we_ref, w1_ref, g1_ref, dinv_ref):
    h = jnp.maximum(jnp.dot(x_ref[...], we_ref[...],
                            preferred_element_type=jnp.float32), 0.0)
    d = deg_ref[0] + deg_ref[1]
    dinv = jnp.where(d > 0, lax.rsqrt(jnp.maximum(d, 1e-12)), 0.0)
    dinv_ref[...] = dinv
    g1_ref[...] = dinv * jnp.dot(h, w1_ref[...],
                                 preferred_element_type=jnp.float32)


def _k2_body(p_ref, w3_ref, dinv_ref, g3_ref):
    h = jnp.maximum(p_ref[0] + p_ref[1], 0.0)
    g3_ref[...] = dinv_ref[...] * jnp.dot(
        h, w3_ref[...], preferred_element_type=jnp.float32)


def _k3_body(q_ref, o_ref):
    o_ref[...] = q_ref[0] + q_ref[1]


_RB = 1024
_GRID = NP // _RB


def _tc_k1(degp, x_pad, We, W1):
    return pl.pallas_call(
        _k1_body,
        grid=(_GRID,),
        in_specs=[
            pl.BlockSpec((2, _RB, 1), lambda i: (0, i, 0)),
            pl.BlockSpec((_RB, D), lambda i: (i, 0)),
            pl.BlockSpec((D, D), lambda i: (0, 0)),
            pl.BlockSpec((D, D), lambda i: (0, 0)),
        ],
        out_specs=[
            pl.BlockSpec((_RB, D), lambda i: (i, 0)),
            pl.BlockSpec((_RB, 1), lambda i: (i, 0)),
        ],
        out_shape=[
            jax.ShapeDtypeStruct((NP, D), jnp.float32),
            jax.ShapeDtypeStruct((NP, 1), jnp.float32),
        ],
    )(degp, x_pad, We, W1)


def _tc_k2(p, W3, dinv_p):
    return pl.pallas_call(
        _k2_body,
        grid=(_GRID,),
        in_specs=[
            pl.BlockSpec((2, _RB, D), lambda i: (0, i, 0)),
            pl.BlockSpec((D, D), lambda i: (0, 0)),
            pl.BlockSpec((_RB, 1), lambda i: (i, 0)),
        ],
        out_specs=pl.BlockSpec((_RB, D), lambda i: (i, 0)),
        out_shape=jax.ShapeDtypeStruct((NP, D), jnp.float32),
    )(p, W3, dinv_p)


def _tc_k3(q):
    return pl.pallas_call(
        _k3_body,
        grid=(_GRID,),
        in_specs=[pl.BlockSpec((2, _RB, D), lambda i: (0, i, 0))],
        out_specs=pl.BlockSpec((_RB, D), lambda i: (i, 0)),
        out_shape=jax.ShapeDtypeStruct((NP, D), jnp.float32),
    )(q)


def kernel(x, edge_index, edge_weight, W_embed, W1, b1, W3, b3):
    row = edge_index[0]
    col = edge_index[1]
    padE = EP - E
    pad_col = (jnp.arange(padE, dtype=jnp.int32) * 8) % N
    row_p = jnp.concatenate([row, jnp.zeros((padE,), jnp.int32)])
    col_p = jnp.concatenate([col, pad_col])
    ew_p = jnp.concatenate([edge_weight, jnp.zeros((padE,), jnp.float32)])
    col_r = col_p.reshape(NW, NB, K)
    ew_r = ew_p.reshape(NW, NB, K)
    x_pad = jnp.pad(x, ((0, NP - N), (0, 0)))
    z1 = jnp.zeros((RPT,), jnp.float32)
    z2 = jnp.zeros((RPT, D), jnp.float32)

    degp = _deg(col_r, ew_r, z1)
    g1, dinv_p = _tc_k1(degp.reshape(2, NP, 1), x_pad, W_embed, W1)
    dinv = dinv_p.reshape(NP)
    p = _conv(g1, dinv, row_p, col_p, ew_p, 0.5 * b1, z2)
    g3 = _tc_k2(p, W3, dinv_p)
    q = _conv(g3, dinv, row_p, col_p, ew_p, 0.5 * b3, z2)
    out = _tc_k3(q)
    return out[:N]


# speedup vs baseline: 1.1580x; 1.0124x over previous
import functools

import jax
import jax.numpy as jnp
from jax import lax
from jax.experimental import pallas as pl
from jax.experimental.pallas import tpu as pltpu
from jax.experimental.pallas import tpu_sc as plsc

N = 10000
E = 320000
D = 128

NC = 2
NS = 16
NW = NC * NS

NP = 10240
RPT = NP // NS
K = 112
NBUF = 3
NB = NBUF * (-(-E // (NW * K * NBUF)))
NB0 = 114
NB1 = 2 * NB - NB0
EPT0 = NB0 * K
EPT1 = NB1 * K
EP = NS * (EPT0 + EPT1)
WCH = 80

_mesh = plsc.VectorSubcoreMesh(core_axis_name="c", subcore_axis_name="s",
                               num_cores=NC, num_subcores=NS)


@functools.partial(
    pl.kernel,
    out_type=jax.ShapeDtypeStruct((NC, NP), jnp.float32),
    mesh=_mesh,
    scratch_types=[
        pltpu.VMEM_SHARED((NP,), jnp.float32),
        pltpu.VMEM((NB, K), jnp.int32),
        pltpu.VMEM((NB, K), jnp.float32),
        pltpu.SemaphoreType.DMA,
    ],
)
def _deg(col_hbm, ew_hbm, z1_hbm, out_hbm, dacc, coli, ewb, sem):
    c = lax.axis_index("c")
    s = lax.axis_index("s")
    wid = s * NC + c
    pltpu.sync_copy(col_hbm.at[wid], coli)
    pltpu.sync_copy(ew_hbm.at[wid], ewb)
    pltpu.sync_copy(z1_hbm, dacc.at[pl.ds(s * RPT, RPT)])
    plsc.subcore_barrier()

    def fire(b, carry):
        pltpu.async_copy(ewb.at[b], dacc.at[coli.at[b]], sem, add=True)
        return carry

    lax.fori_loop(0, NB, fire, 0)

    def drain(b, carry):
        pltpu.make_async_copy(ewb.at[0], dacc.at[coli.at[0]], sem).wait()
        return carry

    lax.fori_loop(0, NB, drain, 0)
    plsc.subcore_barrier()
    pltpu.sync_copy(dacc.at[pl.ds(s * RPT, RPT)],
                    out_hbm.at[c, pl.ds(s * RPT, RPT)])


@functools.partial(
    pl.kernel,
    out_type=jax.ShapeDtypeStruct((NC, NP, D), jnp.float32),
    mesh=_mesh,
    scratch_types=[
        pltpu.VMEM_SHARED((NP, D), jnp.float32),
        pltpu.VMEM((NBUF, K), jnp.int32),
        pltpu.VMEM((NBUF, K), jnp.int32),
        pltpu.VMEM((NBUF, K), jnp.float32),
        pltpu.VMEM((NBUF, K, D), jnp.float32),
        pltpu.VMEM((D,), jnp.float32),
        pltpu.VMEM((WCH,), jnp.float32),
        pltpu.SemaphoreType.DMA((NBUF,)),
        pltpu.SemaphoreType.DMA((NBUF,)),
        pltpu.SemaphoreType.DMA((NBUF,)),
    ],
)
def _conv(g_hbm, dinv_hbm, row_hbm, col_hbm, ew_hbm, bh_hbm, z2_hbm, out_hbm,
          acc, rowi, coli, ewb, rowsb, b_v, dc_v, isem, gsem, ssem):
    c = lax.axis_index("c")
    s = lax.axis_index("s")
    tbase = jnp.where(c == 0, s * EPT0, NS * EPT0 + s * EPT1)
    nb = jnp.where(c == 0, NB0, NB1)
    pltpu.sync_copy(bh_hbm, b_v)
    pltpu.sync_copy(z2_hbm, acc.at[pl.ds(s * RPT, RPT)])

    def issue_idx(b, i):
        base = tbase + b * K
        pltpu.async_copy(row_hbm.at[pl.ds(base, K)], rowi.at[i], isem.at[i])
        pltpu.async_copy(col_hbm.at[pl.ds(base, K)], coli.at[i], isem.at[i])
        pltpu.async_copy(ew_hbm.at[pl.ds(base, K)], ewb.at[i], isem.at[i])

    def wait_idx(i):
        pltpu.make_async_copy(row_hbm.at[pl.ds(0, K)], rowi.at[i],
                              isem.at[i]).wait()
        pltpu.make_async_copy(col_hbm.at[pl.ds(0, K)], coli.at[i],
                              isem.at[i]).wait()
        pltpu.make_async_copy(ew_hbm.at[pl.ds(0, K)], ewb.at[i],
                              isem.at[i]).wait()

    def issue_gather(i):
        pltpu.async_copy(g_hbm.at[rowi.at[i]], rowsb.at[i], gsem.at[i])

    def wait_gather(i):
        pltpu.make_async_copy(g_hbm.at[rowi.at[0]], rowsb.at[i],
                              gsem.at[i]).wait()

    def issue_scatter(i):
        pltpu.async_copy(rowsb.at[i], acc.at[coli.at[i]], ssem.at[i],
                         add=True)

    def wait_scatter(i):
        pltpu.make_async_copy(rowsb.at[i], acc.at[coli.at[0]],
                              ssem.at[i]).wait()

    def scale(i):
        def group(t, carry2):
            w = ewb[i, pl.ds(t * 16, 16)]
            for u in range(16):
                fv = jnp.full((16,), w[u], dtype=jnp.float32)
                k = t * 16 + u
                for j in range(D // 16):
                    sl = pl.ds(j * 16, 16)
                    rowsb[i, k, sl] = rowsb[i, k, sl] * fv
            return carry2

        lax.fori_loop(0, K // 16, group, 0)

    if NBUF == 4:
        issue_idx(0, 0)
        issue_idx(1, 1)
        issue_idx(2, 2)
        plsc.subcore_barrier()
        wait_idx(0)
        issue_gather(0)
        wait_idx(1)
        issue_gather(1)

        def outer(go, carry):
            for i in range(NBUF):
                b = go * NBUF + i
                j2 = (i + 2) % NBUF
                j3 = (i + 3) % NBUF

                @pl.when(b + 2 < nb)
                def _():
                    wait_idx(j2)
                    issue_gather(j2)

                wait_gather(i)
                scale(i)
                issue_scatter(i)

                @pl.when(b + 3 < nb)
                def _():
                    @pl.when(b >= 1)
                    def _():
                        wait_scatter(j3)

                    issue_idx(b + 3, j3)

            return carry
    else:
        issue_idx(0, 0)
        issue_idx(1, 1)
        plsc.subcore_barrier()
        wait_idx(0)
        issue_gather(0)

        def outer(go, carry):
            for i in range(NBUF):
                b = go * NBUF + i
                j1 = (i + 1) % NBUF
                j2 = (i + 2) % NBUF

                @pl.when(b + 1 < nb)
                def _():
                    wait_idx(j1)
                    issue_gather(j1)

                wait_gather(i)
                scale(i)
                issue_scatter(i)

                @pl.when(b + 2 < nb)
                def _():
                    @pl.when(b >= 1)
                    def _():
                        wait_scatter(j2)

                    issue_idx(b + 2, j2)

            return carry

    lax.fori_loop(0, nb // NBUF, outer, 0)
    for i in range(NBUF):
        wait_scatter(i)
    plsc.subcore_barrier()

    for ch in range(RPT // WCH):
        r0 = s * RPT + ch * WCH
        pltpu.sync_copy(acc.at[pl.ds(r0, WCH)], rowsb.at[0, pl.ds(0, WCH)])
        pltpu.sync_copy(dinv_hbm.at[pl.ds(r0, WCH)], dc_v)

        def wgroup(t, carry):
            dvec = dc_v[pl.ds(t * 16, 16)]
            for u in range(16):
                dv = jnp.full((16,), dvec[u], dtype=jnp.float32)
                k = t * 16 + u
                for j in range(D // 16):
                    sl = pl.ds(j * 16, 16)
                    rowsb[0, k, sl] = rowsb[0, k, sl] * dv + b_v[sl]
            return carry

        lax.fori_loop(0, WCH // 16, wgroup, 0)
        pltpu.sync_copy(rowsb.at[0, pl.ds(0, WCH)],
                        out_hbm.at[c, pl.ds(r0, WCH)])


def _k1_body(deg_ref, x_ref, we_ref, w1_ref, g1_ref, dinv_ref):
    h = jnp.maximum(jnp.dot(x_ref[...], we_ref[...],
                            preferred_element_type=jnp.float32), 0.0)
    d = deg_ref[0] + deg_ref[1]
    dinv = jnp.where(d > 0, lax.rsqrt(jnp.maximum(d, 1e-12)), 0.0)
    dinv_ref[...] = dinv
    g1_ref[...] = dinv * jnp.dot(h, w1_ref[...],
                                 preferred_element_type=jnp.float32)


def _k2_body(p_ref, w3_ref, dinv_ref, g3_ref):
    h = jnp.maximum(p_ref[0] + p_ref[1], 0.0)
    g3_ref[...] = dinv_ref[...] * jnp.dot(
        h, w3_ref[...], preferred_element_type=jnp.float32)


def _k3_body(q_ref, o_ref):
    o_ref[...] = q_ref[0] + q_ref[1]


_RB = 1024
_GRID = NP // _RB


def _tc_k1(degp, x_pad, We, W1):
    return pl.pallas_call(
        _k1_body,
        grid=(_GRID,),
        in_specs=[
            pl.BlockSpec((2, _RB, 1), lambda i: (0, i, 0)),
            pl.BlockSpec((_RB, D), lambda i: (i, 0)),
            pl.BlockSpec((D, D), lambda i: (0, 0)),
            pl.BlockSpec((D, D), lambda i: (0, 0)),
        ],
        out_specs=[
            pl.BlockSpec((_RB, D), lambda i: (i, 0)),
            pl.BlockSpec((_RB, 1), lambda i: (i, 0)),
        ],
        out_shape=[
            jax.ShapeDtypeStruct((NP, D), jnp.float32),
            jax.ShapeDtypeStruct((NP, 1), jnp.float32),
        ],
    )(degp, x_pad, We, W1)


def _tc_k2(p, W3, dinv_p):
    return pl.pallas_call(
        _k2_body,
        grid=(_GRID,),
        in_specs=[
            pl.BlockSpec((2, _RB, D), lambda i: (0, i, 0)),
            pl.BlockSpec((D, D), lambda i: (0, 0)),
            pl.BlockSpec((_RB, 1), lambda i: (i, 0)),
        ],
        out_specs=pl.BlockSpec((_RB, D), lambda i: (i, 0)),
        out_shape=jax.ShapeDtypeStruct((NP, D), jnp.float32),
    )(p, W3, dinv_p)


def _tc_k3(q):
    return pl.pallas_call(
        _k3_body,
        grid=(_GRID,),
        in_specs=[pl.BlockSpec((2, _RB, D), lambda i: (0, i, 0))],
        out_specs=pl.BlockSpec((_RB, D), lambda i: (i, 0)),
        out_shape=jax.ShapeDtypeStruct((NP, D), jnp.float32),
    )(q)


def kernel(x, edge_index, edge_weight, W_embed, W1, b1, W3, b3):
    row = edge_index[0]
    col = edge_index[1]
    padE = EP - E
    pad_col = (jnp.arange(padE, dtype=jnp.int32) * 8) % N
    row_p = jnp.concatenate([row, jnp.zeros((padE,), jnp.int32)])
    col_p = jnp.concatenate([col, pad_col])
    ew_p = jnp.concatenate([edge_weight, jnp.zeros((padE,), jnp.float32)])
    col_r = col_p.reshape(NW, NB, K)
    ew_r = ew_p.reshape(NW, NB, K)
    x_pad = jnp.pad(x, ((0, NP - N), (0, 0)))
    z1 = jnp.zeros((RPT,), jnp.float32)
    z2 = jnp.zeros((RPT, D), jnp.float32)

    degp = _deg(col_r, ew_r, z1)
    g1, dinv_p = _tc_k1(degp.reshape(2, NP, 1), x_pad, W_embed, W1)
    dinv = dinv_p.reshape(NP)
    p = _conv(g1, dinv, row_p, col_p, ew_p, 0.5 * b1, z2)
    g3 = _tc_k2(p, W3, dinv_p)
    q = _conv(g3, dinv, row_p, col_p, ew_p, 0.5 * b3, z2)
    out = _tc_k3(q)
    return out[:N]


# speedup vs baseline: 1.1704x; 1.0107x over previous
import functools

import jax
import jax.numpy as jnp
from jax import lax
from jax.experimental import pallas as pl
from jax.experimental.pallas import tpu as pltpu
from jax.experimental.pallas import tpu_sc as plsc

N = 10000
E = 320000
D = 128

NC = 2
NS = 16
NW = NC * NS

NP = 10240
RPT = NP // NS
K = 112
NBUF = 3
NB = NBUF * (-(-E // (NW * K * NBUF)))
NB0 = 117
NB1 = 2 * NB - NB0
EPT0 = NB0 * K
EPT1 = NB1 * K
EP = NS * (EPT0 + EPT1)
WCH = 80

_mesh = plsc.VectorSubcoreMesh(core_axis_name="c", subcore_axis_name="s",
                               num_cores=NC, num_subcores=NS)


@functools.partial(
    pl.kernel,
    out_type=jax.ShapeDtypeStruct((NC, NP), jnp.float32),
    mesh=_mesh,
    scratch_types=[
        pltpu.VMEM_SHARED((NP,), jnp.float32),
        pltpu.VMEM((NB, K), jnp.int32),
        pltpu.VMEM((NB, K), jnp.float32),
        pltpu.SemaphoreType.DMA,
    ],
)
def _deg(col_hbm, ew_hbm, z1_hbm, out_hbm, dacc, coli, ewb, sem):
    c = lax.axis_index("c")
    s = lax.axis_index("s")
    wid = s * NC + c
    pltpu.sync_copy(col_hbm.at[wid], coli)
    pltpu.sync_copy(ew_hbm.at[wid], ewb)
    pltpu.sync_copy(z1_hbm, dacc.at[pl.ds(s * RPT, RPT)])
    plsc.subcore_barrier()

    def fire(b, carry):
        pltpu.async_copy(ewb.at[b], dacc.at[coli.at[b]], sem, add=True)
        return carry

    lax.fori_loop(0, NB, fire, 0)

    def drain(b, carry):
        pltpu.make_async_copy(ewb.at[0], dacc.at[coli.at[0]], sem).wait()
        return carry

    lax.fori_loop(0, NB, drain, 0)
    plsc.subcore_barrier()
    pltpu.sync_copy(dacc.at[pl.ds(s * RPT, RPT)],
                    out_hbm.at[c, pl.ds(s * RPT, RPT)])


@functools.partial(
    pl.kernel,
    out_type=jax.ShapeDtypeStruct((NC, NP, D), jnp.float32),
    mesh=_mesh,
    scratch_types=[
        pltpu.VMEM_SHARED((NP, D), jnp.float32),
        pltpu.VMEM((NBUF, K), jnp.int32),
        pltpu.VMEM((NBUF, K), jnp.int32),
        pltpu.VMEM((NBUF, K), jnp.float32),
        pltpu.VMEM((NBUF, K, D), jnp.float32),
        pltpu.VMEM((D,), jnp.float32),
        pltpu.VMEM((WCH,), jnp.float32),
        pltpu.SemaphoreType.DMA((NBUF,)),
        pltpu.SemaphoreType.DMA((NBUF,)),
        pltpu.SemaphoreType.DMA((NBUF,)),
    ],
)
def _conv(g_hbm, dinv_hbm, row_hbm, col_hbm, ew_hbm, bh_hbm, z2_hbm, out_hbm,
          acc, rowi, coli, ewb, rowsb, b_v, dc_v, isem, gsem, ssem):
    c = lax.axis_index("c")
    s = lax.axis_index("s")
    tbase = jnp.where(c == 0, s * EPT0, NS * EPT0 + s * EPT1)
    nb = jnp.where(c == 0, NB0, NB1)
    pltpu.sync_copy(bh_hbm, b_v)
    pltpu.sync_copy(z2_hbm, acc.at[pl.ds(s * RPT, RPT)])

    def issue_idx(b, i):
        base = tbase + b * K
        pltpu.async_copy(row_hbm.at[pl.ds(base, K)], rowi.at[i], isem.at[i])
        pltpu.async_copy(col_hbm.at[pl.ds(base, K)], coli.at[i], isem.at[i])
        pltpu.async_copy(ew_hbm.at[pl.ds(base, K)], ewb.at[i], isem.at[i])

    def wait_idx(i):
        pltpu.make_async_copy(row_hbm.at[pl.ds(0, K)], rowi.at[i],
                              isem.at[i]).wait()
        pltpu.make_async_copy(col_hbm.at[pl.ds(0, K)], coli.at[i],
                              isem.at[i]).wait()
        pltpu.make_async_copy(ew_hbm.at[pl.ds(0, K)], ewb.at[i],
                              isem.at[i]).wait()

    def issue_gather(i):
        pltpu.async_copy(g_hbm.at[rowi.at[i]], rowsb.at[i], gsem.at[i])

    def wait_gather(i):
        pltpu.make_async_copy(g_hbm.at[rowi.at[0]], rowsb.at[i],
                              gsem.at[i]).wait()

    def issue_scatter(i):
        pltpu.async_copy(rowsb.at[i], acc.at[coli.at[i]], ssem.at[i],
                         add=True)

    def wait_scatter(i):
        pltpu.make_async_copy(rowsb.at[i], acc.at[coli.at[0]],
                              ssem.at[i]).wait()

    def scale(i):
        def group(t, carry2):
            w = ewb[i, pl.ds(t * 16, 16)]
            for u in range(16):
                fv = jnp.full((16,), w[u], dtype=jnp.float32)
                k = t * 16 + u
                for j in range(D // 16):
                    sl = pl.ds(j * 16, 16)
                    rowsb[i, k, sl] = rowsb[i, k, sl] * fv
            return carry2

        lax.fori_loop(0, K // 16, group, 0)

    if NBUF == 4:
        issue_idx(0, 0)
        issue_idx(1, 1)
        issue_idx(2, 2)
        plsc.subcore_barrier()
        wait_idx(0)
        issue_gather(0)
        wait_idx(1)
        issue_gather(1)

        def outer(go, carry):
            for i in range(NBUF):
                b = go * NBUF + i
                j2 = (i + 2) % NBUF
                j3 = (i + 3) % NBUF

                @pl.when(b + 2 < nb)
                def _():
                    wait_idx(j2)
                    issue_gather(j2)

                wait_gather(i)
                scale(i)
                issue_scatter(i)

                @pl.when(b + 3 < nb)
                def _():
                    @pl.when(b >= 1)
                    def _():
                        wait_scatter(j3)

                    issue_idx(b + 3, j3)

            return carry
    else:
        issue_idx(0, 0)
        issue_idx(1, 1)
        plsc.subcore_barrier()
        wait_idx(0)
        issue_gather(0)

        def outer(go, carry):
            for i in range(NBUF):
                b = go * NBUF + i
                j1 = (i + 1) % NBUF
                j2 = (i + 2) % NBUF

                @pl.when(b + 1 < nb)
                def _():
                    wait_idx(j1)
                    issue_gather(j1)

                wait_gather(i)
                scale(i)
                issue_scatter(i)

                @pl.when(b + 2 < nb)
                def _():
                    @pl.when(b >= 1)
                    def _():
                        wait_scatter(j2)

                    issue_idx(b + 2, j2)

            return carry

    lax.fori_loop(0, nb // NBUF, outer, 0)
    for i in range(NBUF):
        wait_scatter(i)
    plsc.subcore_barrier()

    for ch in range(RPT // WCH):
        r0 = s * RPT + ch * WCH
        pltpu.sync_copy(acc.at[pl.ds(r0, WCH)], rowsb.at[0, pl.ds(0, WCH)])
        pltpu.sync_copy(dinv_hbm.at[pl.ds(r0, WCH)], dc_v)

        def wgroup(t, carry):
            dvec = dc_v[pl.ds(t * 16, 16)]
            for u in range(16):
                dv = jnp.full((16,), dvec[u], dtype=jnp.float32)
                k = t * 16 + u
                for j in range(D // 16):
                    sl = pl.ds(j * 16, 16)
                    rowsb[0, k, sl] = rowsb[0, k, sl] * dv + b_v[sl]
            return carry

        lax.fori_loop(0, WCH // 16, wgroup, 0)
        pltpu.sync_copy(rowsb.at[0, pl.ds(0, WCH)],
                        out_hbm.at[c, pl.ds(r0, WCH)])


def _k1_body(deg_ref, x_ref, we_ref, w1_ref, g1_ref, dinv_ref):
    h = jnp.maximum(jnp.dot(x_ref[...], we_ref[...],
                            preferred_element_type=jnp.float32), 0.0)
    d = deg_ref[0] + deg_ref[1]
    dinv = jnp.where(d > 0, lax.rsqrt(jnp.maximum(d, 1e-12)), 0.0)
    dinv_ref[...] = dinv
    g1_ref[...] = dinv * jnp.dot(h, w1_ref[...],
                                 preferred_element_type=jnp.float32)


def _k2_body(p_ref, w3_ref, dinv_ref, g3_ref):
    h = jnp.maximum(p_ref[0] + p_ref[1], 0.0)
    g3_ref[...] = dinv_ref[...] * jnp.dot(
        h, w3_ref[...], preferred_element_type=jnp.float32)


def _k3_body(q_ref, o_ref):
    o_ref[...] = q_ref[0] + q_ref[1]


_RB = 1024
_GRID = NP // _RB


def _tc_k1(degp, x_pad, We, W1):
    return pl.pallas_call(
        _k1_body,
        grid=(_GRID,),
        in_specs=[
            pl.BlockSpec((2, _RB, 1), lambda i: (0, i, 0)),
            pl.BlockSpec((_RB, D), lambda i: (i, 0)),
            pl.BlockSpec((D, D), lambda i: (0, 0)),
            pl.BlockSpec((D, D), lambda i: (0, 0)),
        ],
        out_specs=[
            pl.BlockSpec((_RB, D), lambda i: (i, 0)),
            pl.BlockSpec((_RB, 1), lambda i: (i, 0)),
        ],
        out_shape=[
            jax.ShapeDtypeStruct((NP, D), jnp.float32),
            jax.ShapeDtypeStruct((NP, 1), jnp.float32),
        ],
    )(degp, x_pad, We, W1)


def _tc_k2(p, W3, dinv_p):
    return pl.pallas_call(
        _k2_body,
        grid=(_GRID,),
        in_specs=[
            pl.BlockSpec((2, _RB, D), lambda i: (0, i, 0)),
            pl.BlockSpec((D, D), lambda i: (0, 0)),
            pl.BlockSpec((_RB, 1), lambda i: (i, 0)),
        ],
        out_specs=pl.BlockSpec((_RB, D), lambda i: (i, 0)),
        out_shape=jax.ShapeDtypeStruct((NP, D), jnp.float32),
    )(p, W3, dinv_p)


def _tc_k3(q):
    return pl.pallas_call(
        _k3_body,
        grid=(_GRID,),
        in_specs=[pl.BlockSpec((2, _RB, D), lambda i: (0, i, 0))],
        out_specs=pl.BlockSpec((_RB, D), lambda i: (i, 0)),
        out_shape=jax.ShapeDtypeStruct((NP, D), jnp.float32),
    )(q)


def kernel(x, edge_index, edge_weight, W_embed, W1, b1, W3, b3):
    row = edge_index[0]
    col = edge_index[1]
    padE = EP - E
    pad_col = (jnp.arange(padE, dtype=jnp.int32) * 8) % N
    row_p = jnp.concatenate([row, jnp.zeros((padE,), jnp.int32)])
    col_p = jnp.concatenate([col, pad_col])
    ew_p = jnp.concatenate([edge_weight, jnp.zeros((padE,), jnp.float32)])
    col_r = col_p.reshape(NW, NB, K)
    ew_r = ew_p.reshape(NW, NB, K)
    x_pad = jnp.pad(x, ((0, NP - N), (0, 0)))
    z1 = jnp.zeros((RPT,), jnp.float32)
    z2 = jnp.zeros((RPT, D), jnp.float32)

    degp = _deg(col_r, ew_r, z1)
    g1, dinv_p = _tc_k1(degp.reshape(2, NP, 1), x_pad, W_embed, W1)
    dinv = dinv_p.reshape(NP)
    p = _conv(g1, dinv, row_p, col_p, ew_p, 0.5 * b1, z2)
    g3 = _tc_k2(p, W3, dinv_p)
    q = _conv(g3, dinv, row_p, col_p, ew_p, 0.5 * b3, z2)
    out = _tc_k3(q)
    return out[:N]


# speedup vs baseline: 1.1831x; 1.0109x over previous
import functools

import jax
import jax.numpy as jnp
from jax import lax
from jax.experimental import pallas as pl
from jax.experimental.pallas import tpu as pltpu
from jax.experimental.pallas import tpu_sc as plsc

N = 10000
E = 320000
D = 128

NC = 2
NS = 16
NW = NC * NS

NP = 10240
RPT = NP // NS
K = 112
NBUF = 3
NB = NBUF * (-(-E // (NW * K * NBUF)))
NB0 = 120
NB1 = 2 * NB - NB0
EPT0 = NB0 * K
EPT1 = NB1 * K
EP = NS * (EPT0 + EPT1)
WCH = 80

_mesh = plsc.VectorSubcoreMesh(core_axis_name="c", subcore_axis_name="s",
                               num_cores=NC, num_subcores=NS)


@functools.partial(
    pl.kernel,
    out_type=jax.ShapeDtypeStruct((NC, NP), jnp.float32),
    mesh=_mesh,
    scratch_types=[
        pltpu.VMEM_SHARED((NP,), jnp.float32),
        pltpu.VMEM((NB, K), jnp.int32),
        pltpu.VMEM((NB, K), jnp.float32),
        pltpu.SemaphoreType.DMA,
    ],
)
def _deg(col_hbm, ew_hbm, z1_hbm, out_hbm, dacc, coli, ewb, sem):
    c = lax.axis_index("c")
    s = lax.axis_index("s")
    wid = s * NC + c
    pltpu.sync_copy(col_hbm.at[wid], coli)
    pltpu.sync_copy(ew_hbm.at[wid], ewb)
    pltpu.sync_copy(z1_hbm, dacc.at[pl.ds(s * RPT, RPT)])
    plsc.subcore_barrier()

    def fire(b, carry):
        pltpu.async_copy(ewb.at[b], dacc.at[coli.at[b]], sem, add=True)
        return carry

    lax.fori_loop(0, NB, fire, 0)

    def drain(b, carry):
        pltpu.make_async_copy(ewb.at[0], dacc.at[coli.at[0]], sem).wait()
        return carry

    lax.fori_loop(0, NB, drain, 0)
    plsc.subcore_barrier()
    pltpu.sync_copy(dacc.at[pl.ds(s * RPT, RPT)],
                    out_hbm.at[c, pl.ds(s * RPT, RPT)])


@functools.partial(
    pl.kernel,
    out_type=jax.ShapeDtypeStruct((NC, NP, D), jnp.float32),
    mesh=_mesh,
    scratch_types=[
        pltpu.VMEM_SHARED((NP, D), jnp.float32),
        pltpu.VMEM((NBUF, K), jnp.int32),
        pltpu.VMEM((NBUF, K), jnp.int32),
        pltpu.VMEM((NBUF, K), jnp.float32),
        pltpu.VMEM((NBUF, K, D), jnp.float32),
        pltpu.VMEM((D,), jnp.float32),
        pltpu.VMEM((WCH,), jnp.float32),
        pltpu.SemaphoreType.DMA((NBUF,)),
        pltpu.SemaphoreType.DMA((NBUF,)),
        pltpu.SemaphoreType.DMA((NBUF,)),
    ],
)
def _conv(g_hbm, dinv_hbm, row_hbm, col_hbm, ew_hbm, bh_hbm, z2_hbm, out_hbm,
          acc, rowi, coli, ewb, rowsb, b_v, dc_v, isem, gsem, ssem):
    c = lax.axis_index("c")
    s = lax.axis_index("s")
    tbase = jnp.where(c == 0, s * EPT0, NS * EPT0 + s * EPT1)
    nb = jnp.where(c == 0, NB0, NB1)
    pltpu.sync_copy(bh_hbm, b_v)
    pltpu.sync_copy(z2_hbm, acc.at[pl.ds(s * RPT, RPT)])

    def issue_idx(b, i):
        base = tbase + b * K
        pltpu.async_copy(row_hbm.at[pl.ds(base, K)], rowi.at[i], isem.at[i])
        pltpu.async_copy(col_hbm.at[pl.ds(base, K)], coli.at[i], isem.at[i])
        pltpu.async_copy(ew_hbm.at[pl.ds(base, K)], ewb.at[i], isem.at[i])

    def wait_idx(i):
        pltpu.make_async_copy(row_hbm.at[pl.ds(0, K)], rowi.at[i],
                              isem.at[i]).wait()
        pltpu.make_async_copy(col_hbm.at[pl.ds(0, K)], coli.at[i],
                              isem.at[i]).wait()
        pltpu.make_async_copy(ew_hbm.at[pl.ds(0, K)], ewb.at[i],
                              isem.at[i]).wait()

    def issue_gather(i):
        pltpu.async_copy(g_hbm.at[rowi.at[i]], rowsb.at[i], gsem.at[i])

    def wait_gather(i):
        pltpu.make_async_copy(g_hbm.at[rowi.at[0]], rowsb.at[i],
                              gsem.at[i]).wait()

    def issue_scatter(i):
        pltpu.async_copy(rowsb.at[i], acc.at[coli.at[i]], ssem.at[i],
                         add=True)

    def wait_scatter(i):
        pltpu.make_async_copy(rowsb.at[i], acc.at[coli.at[0]],
                              ssem.at[i]).wait()

    def scale(i):
        def group(t, carry2):
            w = ewb[i, pl.ds(t * 16, 16)]
            for u in range(16):
                fv = jnp.full((16,), w[u], dtype=jnp.float32)
                k = t * 16 + u
                for j in range(D // 16):
                    sl = pl.ds(j * 16, 16)
                    rowsb[i, k, sl] = rowsb[i, k, sl] * fv
            return carry2

        lax.fori_loop(0, K // 16, group, 0)

    if NBUF == 4:
        issue_idx(0, 0)
        issue_idx(1, 1)
        issue_idx(2, 2)
        plsc.subcore_barrier()
        wait_idx(0)
        issue_gather(0)
        wait_idx(1)
        issue_gather(1)

        def outer(go, carry):
            for i in range(NBUF):
                b = go * NBUF + i
                j2 = (i + 2) % NBUF
                j3 = (i + 3) % NBUF

                @pl.when(b + 2 < nb)
                def _():
                    wait_idx(j2)
                    issue_gather(j2)

                wait_gather(i)
                scale(i)
                issue_scatter(i)

                @pl.when(b + 3 < nb)
                def _():
                    @pl.when(b >= 1)
                    def _():
                        wait_scatter(j3)

                    issue_idx(b + 3, j3)

            return carry
    else:
        issue_idx(0, 0)
        issue_idx(1, 1)
        plsc.subcore_barrier()
        wait_idx(0)
        issue_gather(0)

        def outer(go, carry):
            for i in range(NBUF):
                b = go * NBUF + i
                j1 = (i + 1) % NBUF
                j2 = (i + 2) % NBUF

                @pl.when(b + 1 < nb)
                def _():
                    wait_idx(j1)
                    issue_gather(j1)

                wait_gather(i)
                scale(i)
                issue_scatter(i)

                @pl.when(b + 2 < nb)
                def _():
                    @pl.when(b >= 1)
                    def _():
                        wait_scatter(j2)

                    issue_idx(b + 2, j2)

            return carry

    lax.fori_loop(0, nb // NBUF, outer, 0)
    for i in range(NBUF):
        wait_scatter(i)
    plsc.subcore_barrier()

    for ch in range(RPT // WCH):
        r0 = s * RPT + ch * WCH
        pltpu.sync_copy(acc.at[pl.ds(r0, WCH)], rowsb.at[0, pl.ds(0, WCH)])
        pltpu.sync_copy(dinv_hbm.at[pl.ds(r0, WCH)], dc_v)

        def wgroup(t, carry):
            dvec = dc_v[pl.ds(t * 16, 16)]
            for u in range(16):
                dv = jnp.full((16,), dvec[u], dtype=jnp.float32)
                k = t * 16 + u
                for j in range(D // 16):
                    sl = pl.ds(j * 16, 16)
                    rowsb[0, k, sl] = rowsb[0, k, sl] * dv + b_v[sl]
            return carry

        lax.fori_loop(0, WCH // 16, wgroup, 0)
        pltpu.sync_copy(rowsb.at[0, pl.ds(0, WCH)],
                        out_hbm.at[c, pl.ds(r0, WCH)])


def _k1_body(deg_ref, x_ref, we_ref, w1_ref, g1_ref, dinv_ref):
    h = jnp.maximum(jnp.dot(x_ref[...], we_ref[...],
                            preferred_element_type=jnp.float32), 0.0)
    d = deg_ref[0] + deg_ref[1]
    dinv = jnp.where(d > 0, lax.rsqrt(jnp.maximum(d, 1e-12)), 0.0)
    dinv_ref[...] = dinv
    g1_ref[...] = dinv * jnp.dot(h, w1_ref[...],
                                 preferred_element_type=jnp.float32)


def _k2_body(p_ref, w3_ref, dinv_ref, g3_ref):
    h = jnp.maximum(p_ref[0] + p_ref[1], 0.0)
    g3_ref[...] = dinv_ref[...] * jnp.dot(
        h, w3_ref[...], preferred_element_type=jnp.float32)


def _k3_body(q_ref, o_ref):
    o_ref[...] = q_ref[0] + q_ref[1]


_RB = 1024
_GRID = NP // _RB


def _tc_k1(degp, x_pad, We, W1):
    return pl.pallas_call(
        _k1_body,
        grid=(_GRID,),
        in_specs=[
            pl.BlockSpec((2, _RB, 1), lambda i: (0, i, 0)),
            pl.BlockSpec((_RB, D), lambda i: (i, 0)),
            pl.BlockSpec((D, D), lambda i: (0, 0)),
            pl.BlockSpec((D, D), lambda i: (0, 0)),
        ],
        out_specs=[
            pl.BlockSpec((_RB, D), lambda i: (i, 0)),
            pl.BlockSpec((_RB, 1), lambda i: (i, 0)),
        ],
        out_shape=[
            jax.ShapeDtypeStruct((NP, D), jnp.float32),
            jax.ShapeDtypeStruct((NP, 1), jnp.float32),
        ],
    )(degp, x_pad, We, W1)


def _tc_k2(p, W3, dinv_p):
    return pl.pallas_call(
        _k2_body,
        grid=(_GRID,),
        in_specs=[
            pl.BlockSpec((2, _RB, D), lambda i: (0, i, 0)),
            pl.BlockSpec((D, D), lambda i: (0, 0)),
            pl.BlockSpec((_RB, 1), lambda i: (i, 0)),
        ],
        out_specs=pl.BlockSpec((_RB, D), lambda i: (i, 0)),
        out_shape=jax.ShapeDtypeStruct((NP, D), jnp.float32),
    )(p, W3, dinv_p)


def _tc_k3(q):
    return pl.pallas_call(
        _k3_body,
        grid=(_GRID,),
        in_specs=[pl.BlockSpec((2, _RB, D), lambda i: (0, i, 0))],
        out_specs=pl.BlockSpec((_RB, D), lambda i: (i, 0)),
        out_shape=jax.ShapeDtypeStruct((NP, D), jnp.float32),
    )(q)


def kernel(x, edge_index, edge_weight, W_embed, W1, b1, W3, b3):
    row = edge_index[0]
    col = edge_index[1]
    padE = EP - E
    pad_col = (jnp.arange(padE, dtype=jnp.int32) * 8) % N
    row_p = jnp.concatenate([row, jnp.zeros((padE,), jnp.int32)])
    col_p = jnp.concatenate([col, pad_col])
    ew_p = jnp.concatenate([edge_weight, jnp.zeros((padE,), jnp.float32)])
    col_r = col_p.reshape(NW, NB, K)
    ew_r = ew_p.reshape(NW, NB, K)
    x_pad = jnp.pad(x, ((0, NP - N), (0, 0)))
    z1 = jnp.zeros((RPT,), jnp.float32)
    z2 = jnp.zeros((RPT, D), jnp.float32)

    degp = _deg(col_r, ew_r, z1)
    g1, dinv_p = _tc_k1(degp.reshape(2, NP, 1), x_pad, W_embed, W1)
    dinv = dinv_p.reshape(NP)
    p = _conv(g1, dinv, row_p, col_p, ew_p, 0.5 * b1, z2)
    g3 = _tc_k2(p, W3, dinv_p)
    q = _conv(g3, dinv, row_p, col_p, ew_p, 0.5 * b3, z2)
    out = _tc_k3(q)
    return out[:N]


# speedup vs baseline: 1.2098x; 1.0226x over previous
import functools

import jax
import jax.numpy as jnp
from jax import lax
from jax.experimental import pallas as pl
from jax.experimental.pallas import tpu as pltpu
from jax.experimental.pallas import tpu_sc as plsc

N = 10000
E = 320000
D = 128

NC = 2
NS = 16
NW = NC * NS

NP = 10240
RPT = NP // NS
K = 112
NBUF = 3
NB = NBUF * (-(-E // (NW * K * NBUF)))
NB0 = 126
NB1 = 2 * NB - NB0
EPT0 = NB0 * K
EPT1 = NB1 * K
EP = NS * (EPT0 + EPT1)
WCH = 80

_mesh = plsc.VectorSubcoreMesh(core_axis_name="c", subcore_axis_name="s",
                               num_cores=NC, num_subcores=NS)


@functools.partial(
    pl.kernel,
    out_type=jax.ShapeDtypeStruct((NC, NP), jnp.float32),
    mesh=_mesh,
    scratch_types=[
        pltpu.VMEM_SHARED((NP,), jnp.float32),
        pltpu.VMEM((NB, K), jnp.int32),
        pltpu.VMEM((NB, K), jnp.float32),
        pltpu.SemaphoreType.DMA,
    ],
)
def _deg(col_hbm, ew_hbm, z1_hbm, out_hbm, dacc, coli, ewb, sem):
    c = lax.axis_index("c")
    s = lax.axis_index("s")
    wid = s * NC + c
    pltpu.sync_copy(col_hbm.at[wid], coli)
    pltpu.sync_copy(ew_hbm.at[wid], ewb)
    pltpu.sync_copy(z1_hbm, dacc.at[pl.ds(s * RPT, RPT)])
    plsc.subcore_barrier()

    def fire(b, carry):
        pltpu.async_copy(ewb.at[b], dacc.at[coli.at[b]], sem, add=True)
        return carry

    lax.fori_loop(0, NB, fire, 0)

    def drain(b, carry):
        pltpu.make_async_copy(ewb.at[0], dacc.at[coli.at[0]], sem).wait()
        return carry

    lax.fori_loop(0, NB, drain, 0)
    plsc.subcore_barrier()
    pltpu.sync_copy(dacc.at[pl.ds(s * RPT, RPT)],
                    out_hbm.at[c, pl.ds(s * RPT, RPT)])


@functools.partial(
    pl.kernel,
    out_type=jax.ShapeDtypeStruct((NC, NP, D), jnp.float32),
    mesh=_mesh,
    scratch_types=[
        pltpu.VMEM_SHARED((NP, D), jnp.float32),
        pltpu.VMEM((NBUF, K), jnp.int32),
        pltpu.VMEM((NBUF, K), jnp.int32),
        pltpu.VMEM((NBUF, K), jnp.float32),
        pltpu.VMEM((NBUF, K, D), jnp.float32),
        pltpu.VMEM((D,), jnp.float32),
        pltpu.VMEM((WCH,), jnp.float32),
        pltpu.SemaphoreType.DMA((NBUF,)),
        pltpu.SemaphoreType.DMA((NBUF,)),
        pltpu.SemaphoreType.DMA((NBUF,)),
    ],
)
def _conv(g_hbm, dinv_hbm, row_hbm, col_hbm, ew_hbm, bh_hbm, z2_hbm, out_hbm,
          acc, rowi, coli, ewb, rowsb, b_v, dc_v, isem, gsem, ssem):
    c = lax.axis_index("c")
    s = lax.axis_index("s")
    tbase = jnp.where(c == 0, s * EPT0, NS * EPT0 + s * EPT1)
    nb = jnp.where(c == 0, NB0, NB1)
    pltpu.sync_copy(bh_hbm, b_v)
    pltpu.sync_copy(z2_hbm, acc.at[pl.ds(s * RPT, RPT)])

    def issue_idx(b, i):
        base = tbase + b * K
        pltpu.async_copy(row_hbm.at[pl.ds(base, K)], rowi.at[i], isem.at[i])
        pltpu.async_copy(col_hbm.at[pl.ds(base, K)], coli.at[i], isem.at[i])
        pltpu.async_copy(ew_hbm.at[pl.ds(base, K)], ewb.at[i], isem.at[i])

    def wait_idx(i):
        pltpu.make_async_copy(row_hbm.at[pl.ds(0, K)], rowi.at[i],
                              isem.at[i]).wait()
        pltpu.make_async_copy(col_hbm.at[pl.ds(0, K)], coli.at[i],
                              isem.at[i]).wait()
        pltpu.make_async_copy(ew_hbm.at[pl.ds(0, K)], ewb.at[i],
                              isem.at[i]).wait()

    def issue_gather(i):
        pltpu.async_copy(g_hbm.at[rowi.at[i]], rowsb.at[i], gsem.at[i])

    def wait_gather(i):
        pltpu.make_async_copy(g_hbm.at[rowi.at[0]], rowsb.at[i],
                              gsem.at[i]).wait()

    def issue_scatter(i):
        pltpu.async_copy(rowsb.at[i], acc.at[coli.at[i]], ssem.at[i],
                         add=True)

    def wait_scatter(i):
        pltpu.make_async_copy(rowsb.at[i], acc.at[coli.at[0]],
                              ssem.at[i]).wait()

    def scale(i):
        def group(t, carry2):
            w = ewb[i, pl.ds(t * 16, 16)]
            for u in range(16):
                fv = jnp.full((16,), w[u], dtype=jnp.float32)
                k = t * 16 + u
                for j in range(D // 16):
                    sl = pl.ds(j * 16, 16)
                    rowsb[i, k, sl] = rowsb[i, k, sl] * fv
            return carry2

        lax.fori_loop(0, K // 16, group, 0)

    if NBUF == 4:
        issue_idx(0, 0)
        issue_idx(1, 1)
        issue_idx(2, 2)
        plsc.subcore_barrier()
        wait_idx(0)
        issue_gather(0)
        wait_idx(1)
        issue_gather(1)

        def outer(go, carry):
            for i in range(NBUF):
                b = go * NBUF + i
                j2 = (i + 2) % NBUF
                j3 = (i + 3) % NBUF

                @pl.when(b + 2 < nb)
                def _():
                    wait_idx(j2)
                    issue_gather(j2)

                wait_gather(i)
                scale(i)
                issue_scatter(i)

                @pl.when(b + 3 < nb)
                def _():
                    @pl.when(b >= 1)
                    def _():
                        wait_scatter(j3)

                    issue_idx(b + 3, j3)

            return carry
    else:
        issue_idx(0, 0)
        issue_idx(1, 1)
        plsc.subcore_barrier()
        wait_idx(0)
        issue_gather(0)

        def outer(go, carry):
            for i in range(NBUF):
                b = go * NBUF + i
                j1 = (i + 1) % NBUF
                j2 = (i + 2) % NBUF

                @pl.when(b + 1 < nb)
                def _():
                    wait_idx(j1)
                    issue_gather(j1)

                wait_gather(i)
                scale(i)
                issue_scatter(i)

                @pl.when(b + 2 < nb)
                def _():
                    @pl.when(b >= 1)
                    def _():
                        wait_scatter(j2)

                    issue_idx(b + 2, j2)

            return carry

    lax.fori_loop(0, nb // NBUF, outer, 0)
    for i in range(NBUF):
        wait_scatter(i)
    plsc.subcore_barrier()

    for ch in range(RPT // WCH):
        r0 = s * RPT + ch * WCH
        pltpu.sync_copy(acc.at[pl.ds(r0, WCH)], rowsb.at[0, pl.ds(0, WCH)])
        pltpu.sync_copy(dinv_hbm.at[pl.ds(r0, WCH)], dc_v)

        def wgroup(t, carry):
            dvec = dc_v[pl.ds(t * 16, 16)]
            for u in range(16):
                dv = jnp.full((16,), dvec[u], dtype=jnp.float32)
                k = t * 16 + u
                for j in range(D // 16):
                    sl = pl.ds(j * 16, 16)
                    rowsb[0, k, sl] = rowsb[0, k, sl] * dv + b_v[sl]
            return carry

        lax.fori_loop(0, WCH // 16, wgroup, 0)
        pltpu.sync_copy(rowsb.at[0, pl.ds(0, WCH)],
                        out_hbm.at[c, pl.ds(r0, WCH)])


def _k1_body(deg_ref, x_ref, we_ref, w1_ref, g1_ref, dinv_ref):
    h = jnp.maximum(jnp.dot(x_ref[...], we_ref[...],
                            preferred_element_type=jnp.float32), 0.0)
    d = deg_ref[0] + deg_ref[1]
    dinv = jnp.where(d > 0, lax.rsqrt(jnp.maximum(d, 1e-12)), 0.0)
    dinv_ref[...] = dinv
    g1_ref[...] = dinv * jnp.dot(h, w1_ref[...],
                                 preferred_element_type=jnp.float32)


def _k2_body(p_ref, w3_ref, dinv_ref, g3_ref):
    h = jnp.maximum(p_ref[0] + p_ref[1], 0.0)
    g3_ref[...] = dinv_ref[...] * jnp.dot(
        h, w3_ref[...], preferred_element_type=jnp.float32)


def _k3_body(q_ref, o_ref):
    o_ref[...] = q_ref[0] + q_ref[1]


_RB = 1024
_GRID = NP // _RB


def _tc_k1(degp, x_pad, We, W1):
    return pl.pallas_call(
        _k1_body,
        grid=(_GRID,),
        in_specs=[
            pl.BlockSpec((2, _RB, 1), lambda i: (0, i, 0)),
            pl.BlockSpec((_RB, D), lambda i: (i, 0)),
            pl.BlockSpec((D, D), lambda i: (0, 0)),
            pl.BlockSpec((D, D), lambda i: (0, 0)),
        ],
        out_specs=[
            pl.BlockSpec((_RB, D), lambda i: (i, 0)),
            pl.BlockSpec((_RB, 1), lambda i: (i, 0)),
        ],
        out_shape=[
            jax.ShapeDtypeStruct((NP, D), jnp.float32),
            jax.ShapeDtypeStruct((NP, 1), jnp.float32),
        ],
    )(degp, x_pad, We, W1)


def _tc_k2(p, W3, dinv_p):
    return pl.pallas_call(
        _k2_body,
        grid=(_GRID,),
        in_specs=[
            pl.BlockSpec((2, _RB, D), lambda i: (0, i, 0)),
            pl.BlockSpec((D, D), lambda i: (0, 0)),
            pl.BlockSpec((_RB, 1), lambda i: (i, 0)),
        ],
        out_specs=pl.BlockSpec((_RB, D), lambda i: (i, 0)),
        out_shape=jax.ShapeDtypeStruct((NP, D), jnp.float32),
    )(p, W3, dinv_p)


def _tc_k3(q):
    return pl.pallas_call(
        _k3_body,
        grid=(_GRID,),
        in_specs=[pl.BlockSpec((2, _RB, D), lambda i: (0, i, 0))],
        out_specs=pl.BlockSpec((_RB, D), lambda i: (i, 0)),
        out_shape=jax.ShapeDtypeStruct((NP, D), jnp.float32),
    )(q)


def kernel(x, edge_index, edge_weight, W_embed, W1, b1, W3, b3):
    row = edge_index[0]
    col = edge_index[1]
    padE = EP - E
    pad_col = (jnp.arange(padE, dtype=jnp.int32) * 8) % N
    row_p = jnp.concatenate([row, jnp.zeros((padE,), jnp.int32)])
    col_p = jnp.concatenate([col, pad_col])
    ew_p = jnp.concatenate([edge_weight, jnp.zeros((padE,), jnp.float32)])
    col_r = col_p.reshape(NW, NB, K)
    ew_r = ew_p.reshape(NW, NB, K)
    x_pad = jnp.pad(x, ((0, NP - N), (0, 0)))
    z1 = jnp.zeros((RPT,), jnp.float32)
    z2 = jnp.zeros((RPT, D), jnp.float32)

    degp = _deg(col_r, ew_r, z1)
    g1, dinv_p = _tc_k1(degp.reshape(2, NP, 1), x_pad, W_embed, W1)
    dinv = dinv_p.reshape(NP)
    p = _conv(g1, dinv, row_p, col_p, ew_p, 0.5 * b1, z2)
    g3 = _tc_k2(p, W3, dinv_p)
    q = _conv(g3, dinv, row_p, col_p, ew_p, 0.5 * b3, z2)
    out = _tc_k3(q)
    return out[:N]


# speedup vs baseline: 1.2471x; 1.0308x over previous
import functools

import jax
import jax.numpy as jnp
from jax import lax
from jax.experimental import pallas as pl
from jax.experimental.pallas import tpu as pltpu
from jax.experimental.pallas import tpu_sc as plsc

N = 10000
E = 320000
D = 128

NC = 2
NS = 16
NW = NC * NS

NP = 10240
RPT = NP // NS
K = 112
NBUF = 3
NB = NBUF * (-(-E // (NW * K * NBUF)))
NB0 = 132
NB1 = 2 * NB - NB0
EPT0 = NB0 * K
EPT1 = NB1 * K
EP = NS * (EPT0 + EPT1)
WCH = 80

_mesh = plsc.VectorSubcoreMesh(core_axis_name="c", subcore_axis_name="s",
                               num_cores=NC, num_subcores=NS)


@functools.partial(
    pl.kernel,
    out_type=jax.ShapeDtypeStruct((NC, NP), jnp.float32),
    mesh=_mesh,
    scratch_types=[
        pltpu.VMEM_SHARED((NP,), jnp.float32),
        pltpu.VMEM((NB, K), jnp.int32),
        pltpu.VMEM((NB, K), jnp.float32),
        pltpu.SemaphoreType.DMA,
    ],
)
def _deg(col_hbm, ew_hbm, z1_hbm, out_hbm, dacc, coli, ewb, sem):
    c = lax.axis_index("c")
    s = lax.axis_index("s")
    wid = s * NC + c
    pltpu.sync_copy(col_hbm.at[wid], coli)
    pltpu.sync_copy(ew_hbm.at[wid], ewb)
    pltpu.sync_copy(z1_hbm, dacc.at[pl.ds(s * RPT, RPT)])
    plsc.subcore_barrier()

    def fire(b, carry):
        pltpu.async_copy(ewb.at[b], dacc.at[coli.at[b]], sem, add=True)
        return carry

    lax.fori_loop(0, NB, fire, 0)

    def drain(b, carry):
        pltpu.make_async_copy(ewb.at[0], dacc.at[coli.at[0]], sem).wait()
        return carry

    lax.fori_loop(0, NB, drain, 0)
    plsc.subcore_barrier()
    pltpu.sync_copy(dacc.at[pl.ds(s * RPT, RPT)],
                    out_hbm.at[c, pl.ds(s * RPT, RPT)])


@functools.partial(
    pl.kernel,
    out_type=jax.ShapeDtypeStruct((NC, NP, D), jnp.float32),
    mesh=_mesh,
    scratch_types=[
        pltpu.VMEM_SHARED((NP, D), jnp.float32),
        pltpu.VMEM((NBUF, K), jnp.int32),
        pltpu.VMEM((NBUF, K), jnp.int32),
        pltpu.VMEM((NBUF, K), jnp.float32),
        pltpu.VMEM((NBUF, K, D), jnp.float32),
        pltpu.VMEM((D,), jnp.float32),
        pltpu.VMEM((WCH,), jnp.float32),
        pltpu.SemaphoreType.DMA((NBUF,)),
        pltpu.SemaphoreType.DMA((NBUF,)),
        pltpu.SemaphoreType.DMA((NBUF,)),
    ],
)
def _conv(g_hbm, dinv_hbm, row_hbm, col_hbm, ew_hbm, bh_hbm, z2_hbm, out_hbm,
          acc, rowi, coli, ewb, rowsb, b_v, dc_v, isem, gsem, ssem):
    c = lax.axis_index("c")
    s = lax.axis_index("s")
    tbase = jnp.where(c == 0, s * EPT0, NS * EPT0 + s * EPT1)
    nb = jnp.where(c == 0, NB0, NB1)
    pltpu.sync_copy(bh_hbm, b_v)
    pltpu.sync_copy(z2_hbm, acc.at[pl.ds(s * RPT, RPT)])

    def issue_idx(b, i):
        base = tbase + b * K
        pltpu.async_copy(row_hbm.at[pl.ds(base, K)], rowi.at[i], isem.at[i])
        pltpu.async_copy(col_hbm.at[pl.ds(base, K)], coli.at[i], isem.at[i])
        pltpu.async_copy(ew_hbm.at[pl.ds(base, K)], ewb.at[i], isem.at[i])

    def wait_idx(i):
        pltpu.make_async_copy(row_hbm.at[pl.ds(0, K)], rowi.at[i],
                              isem.at[i]).wait()
        pltpu.make_async_copy(col_hbm.at[pl.ds(0, K)], coli.at[i],
                              isem.at[i]).wait()
        pltpu.make_async_copy(ew_hbm.at[pl.ds(0, K)], ewb.at[i],
                              isem.at[i]).wait()

    def issue_gather(i):
        pltpu.async_copy(g_hbm.at[rowi.at[i]], rowsb.at[i], gsem.at[i])

    def wait_gather(i):
        pltpu.make_async_copy(g_hbm.at[rowi.at[0]], rowsb.at[i],
                              gsem.at[i]).wait()

    def issue_scatter(i):
        pltpu.async_copy(rowsb.at[i], acc.at[coli.at[i]], ssem.at[i],
                         add=True)

    def wait_scatter(i):
        pltpu.make_async_copy(rowsb.at[i], acc.at[coli.at[0]],
                              ssem.at[i]).wait()

    def scale(i):
        def group(t, carry2):
            w = ewb[i, pl.ds(t * 16, 16)]
            for u in range(16):
                fv = jnp.full((16,), w[u], dtype=jnp.float32)
                k = t * 16 + u
                for j in range(D // 16):
                    sl = pl.ds(j * 16, 16)
                    rowsb[i, k, sl] = rowsb[i, k, sl] * fv
            return carry2

        lax.fori_loop(0, K // 16, group, 0)

    if NBUF == 4:
        issue_idx(0, 0)
        issue_idx(1, 1)
        issue_idx(2, 2)
        plsc.subcore_barrier()
        wait_idx(0)
        issue_gather(0)
        wait_idx(1)
        issue_gather(1)

        def outer(go, carry):
            for i in range(NBUF):
                b = go * NBUF + i
                j2 = (i + 2) % NBUF
                j3 = (i + 3) % NBUF

                @pl.when(b + 2 < nb)
                def _():
                    wait_idx(j2)
                    issue_gather(j2)

                wait_gather(i)
                scale(i)
                issue_scatter(i)

                @pl.when(b + 3 < nb)
                def _():
                    @pl.when(b >= 1)
                    def _():
                        wait_scatter(j3)

                    issue_idx(b + 3, j3)

            return carry
    else:
        issue_idx(0, 0)
        issue_idx(1, 1)
        plsc.subcore_barrier()
        wait_idx(0)
        issue_gather(0)

        def outer(go, carry):
            for i in range(NBUF):
                b = go * NBUF + i
                j1 = (i + 1) % NBUF
                j2 = (i + 2) % NBUF

                @pl.when(b + 1 < nb)
                def _():
                    wait_idx(j1)
                    issue_gather(j1)

                wait_gather(i)
                scale(i)
                issue_scatter(i)

                @pl.when(b + 2 < nb)
                def _():
                    @pl.when(b >= 1)
                    def _():
                        wait_scatter(j2)

                    issue_idx(b + 2, j2)

            return carry

    lax.fori_loop(0, nb // NBUF, outer, 0)
    for i in range(NBUF):
        wait_scatter(i)
    plsc.subcore_barrier()

    for ch in range(RPT // WCH):
        r0 = s * RPT + ch * WCH
        pltpu.sync_copy(acc.at[pl.ds(r0, WCH)], rowsb.at[0, pl.ds(0, WCH)])
        pltpu.sync_copy(dinv_hbm.at[pl.ds(r0, WCH)], dc_v)

        def wgroup(t, carry):
            dvec = dc_v[pl.ds(t * 16, 16)]
            for u in range(16):
                dv = jnp.full((16,), dvec[u], dtype=jnp.float32)
                k = t * 16 + u
                for j in range(D // 16):
                    sl = pl.ds(j * 16, 16)
                    rowsb[0, k, sl] = rowsb[0, k, sl] * dv + b_v[sl]
            return carry

        lax.fori_loop(0, WCH // 16, wgroup, 0)
        pltpu.sync_copy(rowsb.at[0, pl.ds(0, WCH)],
                        out_hbm.at[c, pl.ds(r0, WCH)])


def _k1_body(deg_ref, x_ref, we_ref, w1_ref, g1_ref, dinv_ref):
    h = jnp.maximum(jnp.dot(x_ref[...], we_ref[...],
                            preferred_element_type=jnp.float32), 0.0)
    d = deg_ref[0] + deg_ref[1]
    dinv = jnp.where(d > 0, lax.rsqrt(jnp.maximum(d, 1e-12)), 0.0)
    dinv_ref[...] = dinv
    g1_ref[...] = dinv * jnp.dot(h, w1_ref[...],
                                 preferred_element_type=jnp.float32)


def _k2_body(p_ref, w3_ref, dinv_ref, g3_ref):
    h = jnp.maximum(p_ref[0] + p_ref[1], 0.0)
    g3_ref[...] = dinv_ref[...] * jnp.dot(
        h, w3_ref[...], preferred_element_type=jnp.float32)


def _k3_body(q_ref, o_ref):
    o_ref[...] = q_ref[0] + q_ref[1]


_RB = 1024
_GRID = NP // _RB


def _tc_k1(degp, x_pad, We, W1):
    return pl.pallas_call(
        _k1_body,
        grid=(_GRID,),
        in_specs=[
            pl.BlockSpec((2, _RB, 1), lambda i: (0, i, 0)),
            pl.BlockSpec((_RB, D), lambda i: (i, 0)),
            pl.BlockSpec((D, D), lambda i: (0, 0)),
            pl.BlockSpec((D, D), lambda i: (0, 0)),
        ],
        out_specs=[
            pl.BlockSpec((_RB, D), lambda i: (i, 0)),
            pl.BlockSpec((_RB, 1), lambda i: (i, 0)),
        ],
        out_shape=[
            jax.ShapeDtypeStruct((NP, D), jnp.float32),
            jax.ShapeDtypeStruct((NP, 1), jnp.float32),
        ],
    )(degp, x_pad, We, W1)


def _tc_k2(p, W3, dinv_p):
    return pl.pallas_call(
        _k2_body,
        grid=(_GRID,),
        in_specs=[
            pl.BlockSpec((2, _RB, D), lambda i: (0, i, 0)),
            pl.BlockSpec((D, D), lambda i: (0, 0)),
            pl.BlockSpec((_RB, 1), lambda i: (i, 0)),
        ],
        out_specs=pl.BlockSpec((_RB, D), lambda i: (i, 0)),
        out_shape=jax.ShapeDtypeStruct((NP, D), jnp.float32),
    )(p, W3, dinv_p)


def _tc_k3(q):
    return pl.pallas_call(
        _k3_body,
        grid=(_GRID,),
        in_specs=[pl.BlockSpec((2, _RB, D), lambda i: (0, i, 0))],
        out_specs=pl.BlockSpec((_RB, D), lambda i: (i, 0)),
        out_shape=jax.ShapeDtypeStruct((NP, D), jnp.float32),
    )(q)


def kernel(x, edge_index, edge_weight, W_embed, W1, b1, W3, b3):
    row = edge_index[0]
    col = edge_index[1]
    padE = EP - E
    pad_col = (jnp.arange(padE, dtype=jnp.int32) * 8) % N
    row_p = jnp.concatenate([row, jnp.zeros((padE,), jnp.int32)])
    col_p = jnp.concatenate([col, pad_col])
    ew_p = jnp.concatenate([edge_weight, jnp.zeros((padE,), jnp.float32)])
    col_r = col_p.reshape(NW, NB, K)
    ew_r = ew_p.reshape(NW, NB, K)
    x_pad = jnp.pad(x, ((0, NP - N), (0, 0)))
    z1 = jnp.zeros((RPT,), jnp.float32)
    z2 = jnp.zeros((RPT, D), jnp.float32)

    degp = _deg(col_r, ew_r, z1)
    g1, dinv_p = _tc_k1(degp.reshape(2, NP, 1), x_pad, W_embed, W1)
    dinv = dinv_p.reshape(NP)
    p = _conv(g1, dinv, row_p, col_p, ew_p, 0.5 * b1, z2)
    g3 = _tc_k2(p, W3, dinv_p)
    q = _conv(g3, dinv, row_p, col_p, ew_p, 0.5 * b3, z2)
    out = _tc_k3(q)
    return out[:N]
